# baseline (speedup 1.0000x reference)
import functools

import jax
import jax.numpy as jnp
from jax import lax
from jax.experimental import pallas as pl
from jax.experimental.pallas import tpu as pltpu
from jax.experimental.pallas import tpu_sc as plsc

MAX_GT = 4096
HALF = MAX_GT // 2
CHUNK = HALF + 16
BUF = CHUNK + 32


def _sc_body(n_lim, w_hbm, counts_hbm, out_hbm, counts_v, buf_v, out_v, sem):
    wid = lax.axis_index("s") * 2 + lax.axis_index("c")
    row = wid // 2
    half = wid % 2

    cdma = pltpu.async_copy(counts_hbm, counts_v, sem)

    neg1 = jnp.full((16,), -1.0, jnp.float32)

    @plsc.parallel_loop(0, HALF, step=16, unroll=8)
    def _fill(i):
        out_v[pl.ds(i, 16)] = neg1

    cdma.wait()
    c = counts_v[...]
    start_i = jnp.int32(0)
    count_i = jnp.int32(0)
    for k in range(16):
        ck = c[k]
        start_i = start_i + jnp.where(k < row, ck, 0)
        count_i = count_i + jnp.where(k == row, ck, 0)

    pbase = half * HALF
    src = start_i + pbase
    w_lo = jnp.minimum((src >> 3) << 3, n_lim)
    w_lo = pl.multiple_of(w_lo, 8)
    delta = src - w_lo
    pltpu.sync_copy(w_hbm.at[pl.ds(w_lo, CHUNK)], buf_v.at[pl.ds(0, CHUNK)])

    rem = jnp.clip(count_i - pbase, 0, HALF)
    ngroups = (rem + 15) >> 4

    lane = lax.iota(jnp.int32, 16)
    vd = delta + lane
    vp = pbase + lane

    nga = jnp.minimum(ngroups, HALF // 32) * 16

    @plsc.parallel_loop(0, nga, step=16, unroll=8)
    def _copy_a(s):
        val = plsc.load_gather(buf_v, [s + vd])
        out_v[pl.ds(s, 16)] = jnp.where(s + vp < count_i, val, neg1)

    odma = pltpu.async_copy(
        out_v.at[pl.ds(0, HALF // 2)], out_hbm.at[pl.ds(wid * HALF, HALF // 2)], sem
    )

    @plsc.parallel_loop(nga, ngroups * 16, step=16, unroll=8)
    def _copy_b(s):
        val = plsc.load_gather(buf_v, [s + vd])
        out_v[pl.ds(s, 16)] = jnp.where(s + vp < count_i, val, neg1)

    pltpu.sync_copy(
        out_v.at[pl.ds(HALF // 2, HALF // 2)],
        out_hbm.at[pl.ds(wid * HALF + HALF // 2, HALF // 2)],
    )
    odma.wait()


def kernel(gt_boxes_select_weight, gt_boxes_batch_ids, batch_num_gt_boxes):
    del gt_boxes_batch_ids
    w = gt_boxes_select_weight
    n_total = w.shape[0]
    b = batch_num_gt_boxes.shape[0]
    counts = batch_num_gt_boxes.reshape(b)

    n_lim = ((n_total - CHUNK) // 8) * 8

    mesh = plsc.VectorSubcoreMesh(core_axis_name="c", subcore_axis_name="s")
    out = pl.kernel(
        functools.partial(_sc_body, n_lim),
        mesh=mesh,
        out_type=jax.ShapeDtypeStruct((b * MAX_GT,), jnp.float32),
        compiler_params=pltpu.CompilerParams(
            needs_layout_passes=False, skip_device_barrier=True
        ),
        scratch_types=[
            pltpu.VMEM((16,), jnp.int32),
            pltpu.VMEM((BUF,), jnp.float32),
            pltpu.VMEM((HALF,), jnp.float32),
            pltpu.SemaphoreType.DMA,
        ],
    )(w, counts)
    return out.reshape(b, MAX_GT)


# speedup vs baseline: 1.0034x; 1.0034x over previous
import functools

import jax
import jax.numpy as jnp
from jax import lax
from jax.experimental import pallas as pl
from jax.experimental.pallas import tpu as pltpu
from jax.experimental.pallas import tpu_sc as plsc

MAX_GT = 4096
HALF = MAX_GT // 2
CHUNK = HALF + 16
BUF = CHUNK + 32


def _sc_body(n_lim, w_hbm, counts_hbm, out_hbm, counts_v, buf_v, out_v, sem):
    wid = lax.axis_index("s") * 2 + lax.axis_index("c")
    row = wid // 2
    half = wid % 2

    cdma = pltpu.async_copy(counts_hbm, counts_v, sem)

    neg1 = jnp.full((16,), -1.0, jnp.float32)

    @plsc.parallel_loop(0, HALF, step=16, unroll=8)
    def _fill(i):
        out_v[pl.ds(i, 16)] = neg1

    cdma.wait()
    c = counts_v[...]
    start_i = jnp.int32(0)
    count_i = jnp.int32(0)
    for k in range(16):
        ck = c[k]
        start_i = start_i + jnp.where(k < row, ck, 0)
        count_i = count_i + jnp.where(k == row, ck, 0)

    pbase = half * HALF
    src = start_i + pbase
    w_lo = jnp.minimum((src >> 3) << 3, n_lim)
    w_lo = pl.multiple_of(w_lo, 8)
    delta = src - w_lo
    pltpu.sync_copy(w_hbm.at[pl.ds(w_lo, CHUNK)], buf_v.at[pl.ds(0, CHUNK)])

    rem = jnp.clip(count_i - pbase, 0, HALF)
    ngroups = (rem + 15) >> 4

    lane = lax.iota(jnp.int32, 16)
    vd = delta + lane
    vp = pbase + lane

    @plsc.parallel_loop(0, ngroups * 16, step=16, unroll=8)
    def _copy(s):
        val = plsc.load_gather(buf_v, [s + vd])
        out_v[pl.ds(s, 16)] = jnp.where(s + vp < count_i, val, neg1)

    pltpu.sync_copy(out_v, out_hbm.at[pl.ds(wid * HALF, HALF)])


def kernel(gt_boxes_select_weight, gt_boxes_batch_ids, batch_num_gt_boxes):
    del gt_boxes_batch_ids
    w = gt_boxes_select_weight
    n_total = w.shape[0]
    b = batch_num_gt_boxes.shape[0]
    counts = batch_num_gt_boxes.reshape(b)

    n_lim = ((n_total - CHUNK) // 8) * 8

    mesh = plsc.VectorSubcoreMesh(core_axis_name="c", subcore_axis_name="s")
    out = pl.kernel(
        functools.partial(_sc_body, n_lim),
        mesh=mesh,
        out_type=jax.ShapeDtypeStruct((b * MAX_GT,), jnp.float32),
        compiler_params=pltpu.CompilerParams(
            needs_layout_passes=False, skip_device_barrier=True
        ),
        scratch_types=[
            pltpu.VMEM((16,), jnp.int32),
            pltpu.VMEM((BUF,), jnp.float32),
            pltpu.VMEM((HALF,), jnp.float32),
            pltpu.SemaphoreType.DMA,
        ],
    )(w, counts)
    return out.reshape(b, MAX_GT)


# speedup vs baseline: 1.0072x; 1.0038x over previous
import functools

import jax
import jax.numpy as jnp
from jax import lax
from jax.experimental import pallas as pl
from jax.experimental.pallas import tpu as pltpu
from jax.experimental.pallas import tpu_sc as plsc

MAX_GT = 4096
HALF = MAX_GT // 2
CHUNK = HALF + 16
BUF = CHUNK + 32


def _sc_body(n_lim, w_hbm, counts_hbm, out_hbm, counts_v, buf_v, out_v, sem):
    wid = lax.axis_index("s") * 2 + lax.axis_index("c")
    row = wid // 2
    half = wid % 2

    cdma = pltpu.async_copy(counts_hbm, counts_v, sem)

    neg1 = jnp.full((16,), -1.0, jnp.float32)

    @plsc.parallel_loop(0, HALF, step=16, unroll=8)
    def _fill(i):
        out_v[pl.ds(i, 16)] = neg1

    cdma.wait()
    c = counts_v[...]
    start_i = jnp.int32(0)
    count_i = jnp.int32(0)
    for k in range(16):
        ck = c[k]
        start_i = start_i + jnp.where(k < row, ck, 0)
        count_i = count_i + jnp.where(k == row, ck, 0)

    pbase = half * HALF
    src = start_i + pbase
    w_lo = jnp.minimum((src >> 3) << 3, n_lim)
    w_lo = pl.multiple_of(w_lo, 8)
    delta = src - w_lo
    pltpu.sync_copy(w_hbm.at[pl.ds(w_lo, CHUNK)], buf_v.at[pl.ds(0, CHUNK)])

    rem = jnp.clip(count_i - pbase, 0, HALF)
    ngroups = (rem + 15) >> 4

    lane = lax.iota(jnp.int32, 16)
    vd = delta + lane
    vp = pbase + lane

    @plsc.parallel_loop(0, ngroups * 16, step=16, unroll=8)
    def _copy(s):
        val = plsc.load_gather(buf_v, [s + vd])
        out_v[pl.ds(s, 16)] = jnp.where(s + vp < count_i, val, neg1)

    pltpu.sync_copy(out_v, out_hbm.at[pl.ds(wid * HALF, HALF)])


def kernel(gt_boxes_select_weight, gt_boxes_batch_ids, batch_num_gt_boxes):
    del gt_boxes_batch_ids
    w = gt_boxes_select_weight
    n_total = w.shape[0]
    b = batch_num_gt_boxes.shape[0]
    counts = batch_num_gt_boxes.reshape(b)

    n_lim = ((n_total - CHUNK) // 8) * 8

    mesh = plsc.VectorSubcoreMesh(core_axis_name="c", subcore_axis_name="s")
    out = pl.kernel(
        functools.partial(_sc_body, n_lim),
        mesh=mesh,
        out_type=jax.ShapeDtypeStruct((b * MAX_GT,), jnp.float32),
        compiler_params=pltpu.CompilerParams(needs_layout_passes=False),
        scratch_types=[
            pltpu.VMEM((16,), jnp.int32),
            pltpu.VMEM((BUF,), jnp.float32),
            pltpu.VMEM((HALF,), jnp.float32),
            pltpu.SemaphoreType.DMA,
        ],
    )(w, counts)
    return out.reshape(b, MAX_GT)
